# Initial kernel scaffold; baseline (speedup 1.0000x reference)
import jax
import jax.numpy as jnp
from jax.experimental import pallas as pl


def kernel(inputs, w0, w, v):
    raise NotImplementedError("write your pallas kernel here")


# speedup vs baseline: 5.8146x; 5.8146x over previous
import functools

import jax
import jax.numpy as jnp
import numpy as np
from jax import lax
from jax.experimental import pallas as pl
from jax.experimental.pallas import tpu as pltpu
from jax.experimental.pallas import tpu_sc as plsc

_FIELD = 26
_FEAT = 10000
_K = 8
_D = _FIELD * _K
_B = 4096
_NC = 2
_NS = 16
_NW = _NC * _NS
_BPW = _B // _NW
_NB = 4
_CIDX = _NB * _FIELD
_NCHUNK = _BPW // _NB
_NVREG = _D // 16

_WPAD = 32
_WIDX = _NB * _WPAD

_OFFS = np.arange(_FIELD, dtype=np.int32) * _FEAT


@functools.partial(
    pl.kernel,
    out_type=jax.ShapeDtypeStruct((_B,), jnp.float32),
    mesh=plsc.VectorSubcoreMesh(
        core_axis_name="c", subcore_axis_name="s", num_cores=_NC, num_subcores=_NS
    ),
    compiler_params=pltpu.CompilerParams(
        needs_layout_passes=False, use_tc_tiling_on_sc=False
    ),
    scratch_types=[
        pltpu.VMEM((_NCHUNK, _CIDX), jnp.int32),
        pltpu.VMEM((_NCHUNK, _WIDX), jnp.int32),
        pltpu.VMEM((_CIDX, _D), jnp.float32),
        pltpu.VMEM((_CIDX, _D), jnp.float32),
        pltpu.VMEM((_WIDX,), jnp.float32),
        pltpu.VMEM((_WIDX,), jnp.float32),
        pltpu.VMEM((_BPW,), jnp.float32),
        pltpu.VMEM((24,), jnp.float32),
        pltpu.SemaphoreType.DMA,
        pltpu.SemaphoreType.DMA,
        pltpu.SemaphoreType.DMA,
        pltpu.SemaphoreType.DMA,
    ],
)
def _ffm_sc(idxv_hbm, idxw_hbm, vtab_hbm, wtab_hbm, out_hbm,
            idxv_l, idxw_l, vb0, vb1, wb0, wb1, out_l, fold,
            sv0, sv1, sw0, sw1):
    wid = lax.axis_index("s") * _NC + lax.axis_index("c")
    base = wid * _BPW

    pltpu.sync_copy(idxv_hbm.at[pl.ds(wid * _NCHUNK, _NCHUNK)], idxv_l)
    pltpu.sync_copy(idxw_hbm.at[pl.ds(wid * _NCHUNK, _NCHUNK)], idxw_l)
    fold[pl.ds(8, 16)] = jnp.zeros((16,), jnp.float32)

    vbufs = (vb0, vb1)
    wbufs = (wb0, wb1)
    svs = (sv0, sv1)
    sws = (sw0, sw1)
    lane_ids = lax.iota(jnp.int32, 16)
    lane_lo = lane_ids < 8
    lane_w = lane_ids < (_FIELD - 16)

    def fire(c, b):
        pltpu.async_copy(vtab_hbm.at[idxv_l.at[c]], vbufs[b], svs[b])
        pltpu.async_copy(wtab_hbm.at[idxw_l.at[c]], wbufs[b], sws[b])

    fire(0, 0)
    fire(1, 1)

    def do_chunk(c, b):
        vb = vbufs[b]
        wb = wbufs[b]
        pltpu.make_async_copy(vtab_hbm.at[idxv_l.at[c]], vb, svs[b]).wait()
        pltpu.make_async_copy(wtab_hbm.at[idxw_l.at[c]], wb, sws[b]).wait()
        for e in range(_NB):
            r0 = e * _FIELD
            accs = tuple(vb[r0, pl.ds(16 * t, 16)] for t in range(_NVREG))

            def red(i, accs):
                r = r0 + 1 + i * 5
                for k in range(5):
                    accs = tuple(
                        accs[t] + vb[r + k, pl.ds(16 * t, 16)]
                        for t in range(_NVREG)
                    )
                return accs

            accs = lax.fori_loop(0, 5, red, accs)

            g = accs[0]
            q = accs[0] * accs[0]
            for t in range(1, _NVREG):
                g = g + accs[t]
                q = q + accs[t] * accs[t]
            sumsq = jnp.sum(q)
            fold[pl.ds(0, 16)] = g
            h = fold[pl.ds(8, 16)]
            s_ext = g + h
            s_m = jnp.where(lane_lo, s_ext, 0.0)
            s2 = jnp.sum(s_m * s_m)
            second = 0.5 * (s2 - sumsq)

            wa = wb[pl.ds(e * _WPAD, 16)]
            wbv = wb[pl.ds(e * _WPAD + 16, 16)]
            ws = jnp.sum(wa + jnp.where(lane_w, wbv, 0.0))
            slot = (c // 4) * 16
            pos = (c % 4) * _NB + e
            cur = out_l[pl.ds(slot, 16)]
            out_l[pl.ds(slot, 16)] = jnp.where(
                lane_ids == pos, ws + second, cur
            )

        @pl.when(c + 2 < _NCHUNK)
        def _():
            fire(c + 2, b)

    def it(i, carry):
        do_chunk(2 * i, 0)
        do_chunk(2 * i + 1, 1)
        return carry

    lax.fori_loop(0, _NCHUNK // 2, it, 0)
    pltpu.sync_copy(out_l, out_hbm.at[pl.ds(base, _BPW)])


@jax.jit
def kernel(inputs, w0, w, v):
    mapped = jnp.asarray(inputs, jnp.int32) + jnp.asarray(_OFFS)[None, :]
    idxv = mapped.reshape(_B * _FIELD // _CIDX, _CIDX)
    idxw = jnp.concatenate(
        [mapped, jnp.zeros((_B, _WPAD - _FIELD), jnp.int32)], axis=1
    ).reshape(_B * _WPAD // _WIDX, _WIDX)
    vflat = v.reshape(_FIELD * _FEAT, _D)
    wflat = w.reshape(_FIELD * _FEAT)
    out = _ffm_sc(idxv, idxw, vflat, wflat)
    return out.reshape(_B, 1) + w0


# speedup vs baseline: 11.9743x; 2.0594x over previous
import functools

import jax
import jax.numpy as jnp
import numpy as np
from jax import lax
from jax.experimental import pallas as pl
from jax.experimental.pallas import tpu as pltpu
from jax.experimental.pallas import tpu_sc as plsc

_FIELD = 26
_FEAT = 10000
_K = 8
_D = _FIELD * _K
_B = 4096
_NC = 2
_NS = 16
_NW = _NC * _NS
_BPW = _B // _NW
_NB = 4
_CIDX = _NB * _FIELD
_NCHUNK = _BPW // _NB
_NVREG = _D // 16

_WPAD = 32
_WIDX = _NB * _WPAD

_OFFS = np.arange(_FIELD, dtype=np.int32) * _FEAT


@functools.partial(
    pl.kernel,
    out_type=jax.ShapeDtypeStruct((_B,), jnp.float32),
    mesh=plsc.VectorSubcoreMesh(
        core_axis_name="c", subcore_axis_name="s", num_cores=_NC, num_subcores=_NS
    ),
    compiler_params=pltpu.CompilerParams(
        needs_layout_passes=False, use_tc_tiling_on_sc=False
    ),
    scratch_types=[
        pltpu.VMEM((_NCHUNK, _CIDX), jnp.int32),
        pltpu.VMEM((_NCHUNK, _WIDX), jnp.int32),
        pltpu.VMEM((_CIDX, _D), jnp.float32),
        pltpu.VMEM((_CIDX, _D), jnp.float32),
        pltpu.VMEM((_WIDX,), jnp.float32),
        pltpu.VMEM((_WIDX,), jnp.float32),
        pltpu.VMEM((_BPW,), jnp.float32),
        pltpu.VMEM((24,), jnp.float32),
        pltpu.SemaphoreType.DMA,
        pltpu.SemaphoreType.DMA,
        pltpu.SemaphoreType.DMA,
        pltpu.SemaphoreType.DMA,
    ],
)
def _ffm_sc(idxv_hbm, idxw_hbm, vtab_hbm, wtab_hbm, out_hbm,
            idxv_l, idxw_l, vb0, vb1, wb0, wb1, out_l, fold,
            sv0, sv1, sw0, sw1):
    wid = lax.axis_index("s") * _NC + lax.axis_index("c")
    base = wid * _BPW

    pltpu.sync_copy(idxv_hbm.at[pl.ds(wid * _NCHUNK, _NCHUNK)], idxv_l)
    pltpu.sync_copy(idxw_hbm.at[pl.ds(wid * _NCHUNK, _NCHUNK)], idxw_l)
    fold[pl.ds(8, 16)] = jnp.zeros((16,), jnp.float32)

    vbufs = (vb0, vb1)
    wbufs = (wb0, wb1)
    svs = (sv0, sv1)
    sws = (sw0, sw1)
    lane_ids = lax.iota(jnp.int32, 16)
    lane_lo = lane_ids < 8
    lane_w = lane_ids < (_FIELD - 16)

    def fire(c, b):
        pltpu.async_copy(vtab_hbm.at[idxv_l.at[c]], vbufs[b], svs[b])
        pltpu.async_copy(wtab_hbm.at[idxw_l.at[c]], wbufs[b], sws[b])

    fire(0, 0)
    fire(1, 1)

    def do_chunk(c, b):
        vb = vbufs[b]
        wb = wbufs[b]
        pltpu.make_async_copy(vtab_hbm.at[idxv_l.at[c]], vb, svs[b]).wait()
        pltpu.make_async_copy(wtab_hbm.at[idxw_l.at[c]], wb, sws[b]).wait()
        for e in range(_NB):
            r0 = e * _FIELD
            accs = tuple(vb[r0, pl.ds(16 * t, 16)] for t in range(_NVREG))

            def red(i, accs):
                r = r0 + 1 + i * 5
                for k in range(5):
                    accs = tuple(
                        accs[t] + vb[r + k, pl.ds(16 * t, 16)]
                        for t in range(_NVREG)
                    )
                return accs

            accs = lax.fori_loop(0, 5, red, accs)

            g = accs[0]
            q = accs[0] * accs[0]
            for t in range(1, _NVREG):
                g = g + accs[t]
                q = q + accs[t] * accs[t]
            sumsq = jnp.sum(q)
            fold[pl.ds(0, 16)] = g
            h = fold[pl.ds(8, 16)]
            s_ext = g + h
            s_m = jnp.where(lane_lo, s_ext, 0.0)
            s2 = jnp.sum(s_m * s_m)
            second = 0.5 * (s2 - sumsq)

            wa = wb[pl.ds(e * _WPAD, 16)]
            wbv = wb[pl.ds(e * _WPAD + 16, 16)]
            ws = jnp.sum(wa + jnp.where(lane_w, wbv, 0.0))
            slot = (c // 4) * 16
            pos = (c % 4) * _NB + e
            cur = out_l[pl.ds(slot, 16)]
            out_l[pl.ds(slot, 16)] = jnp.where(
                lane_ids == pos, ws + second, cur
            )

        @pl.when(c + 2 < _NCHUNK)
        def _():
            fire(c + 2, b)

    def it(i, carry):
        do_chunk(2 * i, 0)
        do_chunk(2 * i + 1, 1)
        return carry

    lax.fori_loop(0, _NCHUNK // 2, it, 0)
    pltpu.sync_copy(out_l, out_hbm.at[pl.ds(base, _BPW)])


_NROW = _FIELD * _FEAT
_TBN = 1024


def _tr_body(x_ref, o_ref):
    o_ref[...] = x_ref[...].T


_transpose_v = pl.pallas_call(
    _tr_body,
    grid=(pl.cdiv(_NROW, _TBN),),
    in_specs=[pl.BlockSpec((_D, _TBN), lambda j: (0, j))],
    out_specs=pl.BlockSpec((_TBN, _D), lambda j: (j, 0)),
    out_shape=jax.ShapeDtypeStruct((_NROW, _D), jnp.float32),
)


@jax.jit
def kernel(inputs, w0, w, v):
    mapped = jnp.asarray(inputs, jnp.int32) + jnp.asarray(_OFFS)[None, :]
    idxv = mapped.reshape(_B * _FIELD // _CIDX, _CIDX)
    idxw = jnp.concatenate(
        [mapped, jnp.zeros((_B, _WPAD - _FIELD), jnp.int32)], axis=1
    ).reshape(_B * _WPAD // _WIDX, _WIDX)
    vflat = _transpose_v(v.reshape(_NROW, _D).T)
    wflat = w.reshape(_NROW)
    out = _ffm_sc(idxv, idxw, vflat, wflat)
    return out.reshape(_B, 1) + w0


# speedup vs baseline: 13.9071x; 1.1614x over previous
import functools

import jax
import jax.numpy as jnp
import numpy as np
from jax import lax
from jax.experimental import pallas as pl
from jax.experimental.pallas import tpu as pltpu
from jax.experimental.pallas import tpu_sc as plsc

_FIELD = 26
_FEAT = 10000
_K = 8
_D = _FIELD * _K
_B = 4096
_NC = 2
_NS = 16
_NW = _NC * _NS
_BPW = _B // _NW
_NB = 4
_CIDX = _NB * _FIELD
_NCHUNK = _BPW // _NB
_NVREG = _D // 16

_WPAD = 32
_WIDX = _NB * _WPAD

_OFFS = np.arange(_FIELD, dtype=np.int32) * _FEAT


@functools.partial(
    pl.kernel,
    out_type=jax.ShapeDtypeStruct((_B,), jnp.float32),
    mesh=plsc.VectorSubcoreMesh(
        core_axis_name="c", subcore_axis_name="s", num_cores=_NC, num_subcores=_NS
    ),
    compiler_params=pltpu.CompilerParams(
        needs_layout_passes=False, use_tc_tiling_on_sc=False
    ),
    scratch_types=[
        pltpu.VMEM((_NCHUNK, _CIDX), jnp.int32),
        pltpu.VMEM((_NCHUNK, _WIDX), jnp.int32),
        pltpu.VMEM((_CIDX, _D), jnp.float32),
        pltpu.VMEM((_CIDX, _D), jnp.float32),
        pltpu.VMEM((_WIDX,), jnp.float32),
        pltpu.VMEM((_WIDX,), jnp.float32),
        pltpu.VMEM((_BPW,), jnp.float32),
        pltpu.VMEM((24,), jnp.float32),
        pltpu.SemaphoreType.DMA,
        pltpu.SemaphoreType.DMA,
        pltpu.SemaphoreType.DMA,
        pltpu.SemaphoreType.DMA,
    ],
)
def _ffm_sc(idxv_hbm, idxw_hbm, vtab_hbm, wtab_hbm, out_hbm,
            idxv_l, idxw_l, vb0, vb1, wb0, wb1, out_l, fold,
            sv0, sv1, sw0, sw1):
    wid = lax.axis_index("s") * _NC + lax.axis_index("c")
    base = wid * _BPW

    pltpu.sync_copy(idxv_hbm.at[pl.ds(wid * _NCHUNK, _NCHUNK)], idxv_l)
    pltpu.sync_copy(idxw_hbm.at[pl.ds(wid * _NCHUNK, _NCHUNK)], idxw_l)
    fold[pl.ds(8, 16)] = jnp.zeros((16,), jnp.float32)

    vbufs = (vb0, vb1)
    wbufs = (wb0, wb1)
    svs = (sv0, sv1)
    sws = (sw0, sw1)
    lane_ids = lax.iota(jnp.int32, 16)
    lane_lo = lane_ids < 8
    lane_w = lane_ids < (_FIELD - 16)

    def fire(c, b):
        pltpu.async_copy(vtab_hbm.at[idxv_l.at[c]], vbufs[b], svs[b])
        pltpu.async_copy(wtab_hbm.at[idxw_l.at[c]], wbufs[b], sws[b])

    fire(0, 0)
    fire(1, 1)

    def do_chunk(c, b):
        vb = vbufs[b]
        wb = wbufs[b]
        pltpu.make_async_copy(vtab_hbm.at[idxv_l.at[c]], vb, svs[b]).wait()
        pltpu.make_async_copy(wtab_hbm.at[idxw_l.at[c]], wb, sws[b]).wait()
        for e in range(_NB):
            r0 = e * _FIELD
            accs = tuple(vb[r0, pl.ds(16 * t, 16)] for t in range(_NVREG))

            def red(i, accs):
                r = r0 + 1 + i * 5
                for k in range(5):
                    accs = tuple(
                        accs[t] + vb[r + k, pl.ds(16 * t, 16)]
                        for t in range(_NVREG)
                    )
                return accs

            accs = lax.fori_loop(0, 5, red, accs)

            g = accs[0]
            q = accs[0] * accs[0]
            for t in range(1, _NVREG):
                g = g + accs[t]
                q = q + accs[t] * accs[t]
            sumsq = jnp.sum(q)
            fold[pl.ds(0, 16)] = g
            h = fold[pl.ds(8, 16)]
            s_ext = g + h
            s_m = jnp.where(lane_lo, s_ext, 0.0)
            s2 = jnp.sum(s_m * s_m)
            second = 0.5 * (s2 - sumsq)

            wa = wb[pl.ds(e * _WPAD, 16)]
            wbv = wb[pl.ds(e * _WPAD + 16, 16)]
            ws = jnp.sum(wa + jnp.where(lane_w, wbv, 0.0))
            slot = (c // 4) * 16
            pos = (c % 4) * _NB + e
            cur = out_l[pl.ds(slot, 16)]
            out_l[pl.ds(slot, 16)] = jnp.where(
                lane_ids == pos, ws + second, cur
            )

        @pl.when(c + 2 < _NCHUNK)
        def _():
            fire(c + 2, b)

    def it(i, carry):
        do_chunk(2 * i, 0)
        do_chunk(2 * i + 1, 1)
        return carry

    lax.fori_loop(0, _NCHUNK // 2, it, 0)
    pltpu.sync_copy(out_l, out_hbm.at[pl.ds(base, _BPW)])


_NROW = _FIELD * _FEAT
_TBN = 4096


def _tr_body(x_ref, o_ref):
    o_ref[...] = x_ref[...].T


_transpose_v = pl.pallas_call(
    _tr_body,
    grid=(pl.cdiv(_NROW, _TBN),),
    in_specs=[pl.BlockSpec((_D, _TBN), lambda j: (0, j))],
    out_specs=pl.BlockSpec((_TBN, _D), lambda j: (j, 0)),
    out_shape=jax.ShapeDtypeStruct((_NROW, _D), jnp.float32),
)


@jax.jit
def kernel(inputs, w0, w, v):
    mapped = jnp.asarray(inputs, jnp.int32) + jnp.asarray(_OFFS)[None, :]
    idxv = mapped.reshape(_B * _FIELD // _CIDX, _CIDX)
    idxw = jnp.concatenate(
        [mapped, jnp.zeros((_B, _WPAD - _FIELD), jnp.int32)], axis=1
    ).reshape(_B * _WPAD // _WIDX, _WIDX)
    vflat = _transpose_v(v.reshape(_NROW, _D).T)
    wflat = w.reshape(_NROW)
    out = _ffm_sc(idxv, idxw, vflat, wflat)
    return out.reshape(_B, 1) + w0


# speedup vs baseline: 14.0464x; 1.0100x over previous
import functools

import jax
import jax.numpy as jnp
import numpy as np
from jax import lax
from jax.experimental import pallas as pl
from jax.experimental.pallas import tpu as pltpu
from jax.experimental.pallas import tpu_sc as plsc

_FIELD = 26
_FEAT = 10000
_K = 8
_D = _FIELD * _K
_B = 4096
_NC = 2
_NS = 16
_NW = _NC * _NS
_BPW = _B // _NW
_NB = 4
_CIDX = _NB * _FIELD
_NCHUNK = _BPW // _NB
_NVREG = _D // 16

_WPAD = 32
_WIDX = _NB * _WPAD

_OFFS = np.arange(_FIELD, dtype=np.int32) * _FEAT


@functools.partial(
    pl.kernel,
    out_type=jax.ShapeDtypeStruct((_B,), jnp.float32),
    mesh=plsc.VectorSubcoreMesh(
        core_axis_name="c", subcore_axis_name="s", num_cores=_NC, num_subcores=_NS
    ),
    compiler_params=pltpu.CompilerParams(
        needs_layout_passes=False, use_tc_tiling_on_sc=False
    ),
    scratch_types=[
        pltpu.VMEM((_NCHUNK, _CIDX), jnp.int32),
        pltpu.VMEM((_NCHUNK, _WIDX), jnp.int32),
        pltpu.VMEM((_CIDX, _D), jnp.float32),
        pltpu.VMEM((_CIDX, _D), jnp.float32),
        pltpu.VMEM((_WIDX,), jnp.float32),
        pltpu.VMEM((_WIDX,), jnp.float32),
        pltpu.VMEM((_BPW,), jnp.float32),
        pltpu.VMEM((24,), jnp.float32),
        pltpu.SemaphoreType.DMA,
        pltpu.SemaphoreType.DMA,
        pltpu.SemaphoreType.DMA,
        pltpu.SemaphoreType.DMA,
    ],
)
def _ffm_sc(idxv_hbm, idxw_hbm, vtab_hbm, wtab_hbm, out_hbm,
            idxv_l, idxw_l, vb0, vb1, wb0, wb1, out_l, fold,
            sv0, sv1, sw0, sw1):
    wid = lax.axis_index("s") * _NC + lax.axis_index("c")
    base = wid * _BPW

    pltpu.sync_copy(idxv_hbm.at[pl.ds(wid * _NCHUNK, _NCHUNK)], idxv_l)
    pltpu.sync_copy(idxw_hbm.at[pl.ds(wid * _NCHUNK, _NCHUNK)], idxw_l)
    fold[pl.ds(8, 16)] = jnp.zeros((16,), jnp.float32)

    vbufs = (vb0, vb1)
    wbufs = (wb0, wb1)
    svs = (sv0, sv1)
    sws = (sw0, sw1)
    lane_ids = lax.iota(jnp.int32, 16)
    lane_lo = lane_ids < 8
    lane_w = lane_ids < (_FIELD - 16)

    def fire(c, b):
        pltpu.async_copy(vtab_hbm.at[idxv_l.at[c]], vbufs[b], svs[b])
        pltpu.async_copy(wtab_hbm.at[idxw_l.at[c]], wbufs[b], sws[b])

    fire(0, 0)
    fire(1, 1)

    def do_chunk(c, b):
        vb = vbufs[b]
        wb = wbufs[b]
        pltpu.make_async_copy(vtab_hbm.at[idxv_l.at[c]], vb, svs[b]).wait()
        pltpu.make_async_copy(wtab_hbm.at[idxw_l.at[c]], wb, sws[b]).wait()
        for e in range(_NB):
            r0 = e * _FIELD
            accs = tuple(vb[r0, pl.ds(16 * t, 16)] for t in range(_NVREG))

            def red(i, accs):
                r = r0 + 1 + i * 5
                for k in range(5):
                    accs = tuple(
                        accs[t] + vb[r + k, pl.ds(16 * t, 16)]
                        for t in range(_NVREG)
                    )
                return accs

            accs = lax.fori_loop(0, 5, red, accs)

            g = accs[0]
            q = accs[0] * accs[0]
            for t in range(1, _NVREG):
                g = g + accs[t]
                q = q + accs[t] * accs[t]
            sumsq = jnp.sum(q)
            fold[pl.ds(0, 16)] = g
            h = fold[pl.ds(8, 16)]
            s_ext = g + h
            s_m = jnp.where(lane_lo, s_ext, 0.0)
            s2 = jnp.sum(s_m * s_m)
            second = 0.5 * (s2 - sumsq)

            wa = wb[pl.ds(e * _WPAD, 16)]
            wbv = wb[pl.ds(e * _WPAD + 16, 16)]
            ws = jnp.sum(wa + jnp.where(lane_w, wbv, 0.0))
            slot = (c // 4) * 16
            pos = (c % 4) * _NB + e
            cur = out_l[pl.ds(slot, 16)]
            out_l[pl.ds(slot, 16)] = jnp.where(
                lane_ids == pos, ws + second, cur
            )

        @pl.when(c + 2 < _NCHUNK)
        def _():
            fire(c + 2, b)

    def it(i, carry):
        do_chunk(2 * i, 0)
        do_chunk(2 * i + 1, 1)
        return carry

    lax.fori_loop(0, _NCHUNK // 2, it, 0)
    pltpu.sync_copy(out_l, out_hbm.at[pl.ds(base, _BPW)])


_NROW = _FIELD * _FEAT
_TBN = 8192


def _tr_body(x_ref, o_ref):
    o_ref[...] = x_ref[...].T


_transpose_v = pl.pallas_call(
    _tr_body,
    grid=(pl.cdiv(_NROW, _TBN),),
    in_specs=[pl.BlockSpec((_D, _TBN), lambda j: (0, j))],
    out_specs=pl.BlockSpec((_TBN, _D), lambda j: (j, 0)),
    out_shape=jax.ShapeDtypeStruct((_NROW, _D), jnp.float32),
)


@jax.jit
def kernel(inputs, w0, w, v):
    mapped = jnp.asarray(inputs, jnp.int32) + jnp.asarray(_OFFS)[None, :]
    idxv = mapped.reshape(_B * _FIELD // _CIDX, _CIDX)
    idxw = jnp.concatenate(
        [mapped, jnp.zeros((_B, _WPAD - _FIELD), jnp.int32)], axis=1
    ).reshape(_B * _WPAD // _WIDX, _WIDX)
    vflat = _transpose_v(v.reshape(_NROW, _D).T)
    wflat = w.reshape(_NROW)
    out = _ffm_sc(idxv, idxw, vflat, wflat)
    return out.reshape(_B, 1) + w0
